# baseline (speedup 1.0000x reference)
import functools

import jax
import jax.numpy as jnp
from jax import lax
from jax.experimental import pallas as pl
from jax.experimental.pallas import tpu as pltpu
from jax.experimental.pallas import tpu_sc as plsc

VOCAB = 1000000
HIDDEN = 64
OUT = 1000
B = 20480

NC = 2
NS = 16
NW = NC * NS

CH = 256
SPAN = 31232
NCH_LO = SPAN // CH
SPAN_HI = VOCAB - 64 - 31 * SPAN
NCH_HI = SPAN_HI // CH
TAIL0 = VOCAB - 64
RING = 64


def _sc_gather_stream(idx, emb_t, tail_t):
    mesh = plsc.VectorSubcoreMesh(core_axis_name="c", subcore_axis_name="s")

    @functools.partial(
        pl.kernel,
        mesh=mesh,
        out_type=jax.ShapeDtypeStruct((B, HIDDEN), jnp.float32),
        scratch_types=[
            pltpu.VMEM((B,), jnp.int32),
            pltpu.VMEM((B,), jnp.int32),
            pltpu.VMEM((B,), jnp.int32),
            pltpu.VMEM((3, HIDDEN, CH), jnp.float32),
            pltpu.VMEM((RING, HIDDEN), jnp.float32),
            pltpu.VMEM((HIDDEN, 64), jnp.float32),
            pltpu.SMEM((4,), jnp.int32),
            pltpu.SemaphoreType.DMA,
            pltpu.SemaphoreType.DMA,
        ],
        compiler_params=pltpu.CompilerParams(needs_layout_passes=False),
    )
    def gather_kernel(idx_hbm, emb_hbm, tail_hbm, out_hbm,
                      idx_v, pos_v, cpos_v, chunk_v, ring_v, tail_v,
                      cnt_s, csem, rsem):
        wid = lax.axis_index("s") * NC + lax.axis_index("c")
        last = wid == NW - 1
        base = wid * SPAN
        himark = jnp.where(last, VOCAB, base + SPAN)
        nch = jnp.where(last, NCH_HI, NCH_LO)
        lane16 = jax.lax.iota(jnp.int32, 16)

        pltpu.sync_copy(idx_hbm, idx_v)
        cnt_s[0] = 0
        cnt_s[1] = 0

        def build(g, n):
            iv = idx_v[pl.ds(g * 16, 16)]
            mm = (iv >= base) & (iv < himark)
            cnt = plsc.all_reduce_population_count(mm)[0]

            @pl.when(cnt > 0)
            def _():
                inc = plsc.cumsum(jnp.where(mm, 1, 0))
                plsc.store_scatter(
                    pos_v, [n + inc - 1], g * 16 + lane16, mask=mm
                )
            return n + cnt

        n_mine = lax.fori_loop(0, B // 16, build, 0)
        ngrp = (n_mine + 15) // 16

        def extract_hits(chunk_ref, clo, m_hits):
            def one_group(g2, _):
                valid = g2 * 16 + lane16 < m_hits
                cp = jnp.where(valid, cpos_v[pl.ds(g2 * 16, 16)], 0)
                iv = plsc.load_gather(idx_v, [cp])
                cols = jnp.where(valid, iv - clo, 0)
                valid_i = jnp.where(valid, 1, 0)
                for lane in range(16):
                    @pl.when(valid_i[lane] == 1)
                    def _():
                        slot = lax.rem(cnt_s[0], RING)
                        col16 = jnp.full((16,), cols[lane], jnp.int32)
                        for j in range(HIDDEN // 16):
                            vals = plsc.load_gather(
                                chunk_ref, [lane16 + j * 16, col16]
                            )
                            ring_v[slot, pl.ds(j * 16, 16)] = vals
                        pltpu.async_copy(
                            ring_v.at[pl.ds(slot, 1)],
                            out_hbm.at[pl.ds(cp[lane], 1)],
                            rsem,
                        )
                        cnt_s[0] = cnt_s[0] + 1
                        cnt_s[1] = cnt_s[1] + 1

                @pl.when(cnt_s[1] >= RING - 16)
                def _():
                    def drain(_, c):
                        pltpu.make_async_copy(
                            out_hbm.at[pl.ds(0, 1)],
                            ring_v.at[pl.ds(0, 1)],
                            rsem,
                        ).wait()
                        return c
                    lax.fori_loop(0, cnt_s[1], drain, 0)
                    cnt_s[1] = 0
                return 0

            lax.fori_loop(0, (m_hits + 15) // 16, one_group, 0)

        def scan_hits(clo, chi):
            def ga(g, m):
                valid = g * 16 + lane16 < n_mine
                pv = jnp.where(valid, pos_v[pl.ds(g * 16, 16)], 0)
                iv = plsc.load_gather(idx_v, [pv])
                mm = valid & (iv >= clo) & (iv < chi)
                cnt = plsc.all_reduce_population_count(mm)[0]

                @pl.when(cnt > 0)
                def _():
                    inc = plsc.cumsum(jnp.where(mm, 1, 0))
                    plsc.store_scatter(cpos_v, [m + inc - 1], pv, mask=mm)
                return m + cnt

            return lax.fori_loop(0, ngrp, ga, 0)

        pltpu.async_copy(
            emb_hbm.at[:, pl.ds(pl.multiple_of(base, 128), CH)],
            chunk_v.at[0], csem,
        )
        @pl.when(1 < nch)
        def _():
            pltpu.async_copy(
                emb_hbm.at[:, pl.ds(pl.multiple_of(base + CH, 128), CH)],
                chunk_v.at[1], csem,
            )

        def chunk_step(c, _):
            @pl.when(c + 2 < nch)
            def _():
                off = base + (c + 2) * CH
                pltpu.async_copy(
                    emb_hbm.at[:, pl.ds(pl.multiple_of(off, 128), CH)],
                    chunk_v.at[lax.rem(c + 2, 3)], csem,
                )
            pltpu.make_async_copy(
                emb_hbm.at[:, pl.ds(0, CH)], chunk_v.at[0], csem
            ).wait()
            clo = base + c * CH
            m_hits = scan_hits(clo, clo + CH)
            extract_hits(chunk_v.at[lax.rem(c, 3)], clo, m_hits)
            return 0

        lax.fori_loop(0, nch, chunk_step, 0)

        @pl.when(last)
        def _():
            pltpu.sync_copy(tail_hbm, tail_v)
            m_hits = scan_hits(TAIL0, VOCAB)
            extract_hits(tail_v, TAIL0, m_hits)

        def drain(_, c):
            pltpu.make_async_copy(
                out_hbm.at[pl.ds(0, 1)], ring_v.at[pl.ds(0, 1)], rsem
            ).wait()
            return c
        lax.fori_loop(0, cnt_s[1], drain, 0)

    return gather_kernel(idx, emb_t, tail_t)


def _tc_decode_t(x_t, w, b2):
    bm = 2048
    grid = (B // bm,)

    def body(w_ref, x_ref, b_ref, o_ref):
        xr = jnp.maximum(x_ref[...], 0.0)
        o_ref[...] = (
            jnp.dot(w_ref[...], xr, preferred_element_type=jnp.float32)
            + b_ref[...]
        )

    return pl.pallas_call(
        body,
        grid=grid,
        in_specs=[
            pl.BlockSpec((OUT, HIDDEN), lambda i: (0, 0)),
            pl.BlockSpec((HIDDEN, bm), lambda i: (0, i)),
            pl.BlockSpec((OUT, 1), lambda i: (0, 0)),
        ],
        out_specs=pl.BlockSpec((OUT, bm), lambda i: (0, i)),
        out_shape=jax.ShapeDtypeStruct((OUT, B), jnp.float32),
    )(w, x_t, b2)


def kernel(inp, hidden, emb, W, b):
    idx = inp.reshape(-1).astype(jnp.int32)
    emb_t = emb.T
    tail_t = lax.slice(emb_t, (0, TAIL0), (HIDDEN, VOCAB))
    x = _sc_gather_stream(idx, emb_t, tail_t)
    logits_t = _tc_decode_t(x.T, W, b.reshape(OUT, 1))
    return (logits_t.T, hidden)


# speedup vs baseline: 1.5356x; 1.5356x over previous
import functools

import jax
import jax.numpy as jnp
from jax import lax
from jax.experimental import pallas as pl
from jax.experimental.pallas import tpu as pltpu
from jax.experimental.pallas import tpu_sc as plsc

VOCAB = 1000000
HIDDEN = 64
OUT = 1000
B = 20480

NC = 2
NS = 16
NW = NC * NS

CH = 256
SPAN = 31232
NCH_LO = SPAN // CH
SPAN_HI = VOCAB - 64 - 31 * SPAN
NCH_HI = SPAN_HI // CH
TAIL0 = VOCAB - 64
RING = 32


def _sc_gather_stream(idx, emb_t, tail_t):
    mesh = plsc.VectorSubcoreMesh(core_axis_name="c", subcore_axis_name="s")

    @functools.partial(
        pl.kernel,
        mesh=mesh,
        out_type=jax.ShapeDtypeStruct((B, HIDDEN), jnp.float32),
        scratch_types=[
            pltpu.VMEM((B,), jnp.int32),
            pltpu.VMEM((B,), jnp.int32),
            pltpu.VMEM((B + 128,), jnp.int32),
            pltpu.VMEM((B,), jnp.int32),
            pltpu.VMEM((2, HIDDEN, CH), jnp.float32),
            pltpu.VMEM((RING, HIDDEN), jnp.float32),
            pltpu.VMEM((HIDDEN, 64), jnp.float32),
            pltpu.SMEM((24,), jnp.int32),
            pltpu.SemaphoreType.DMA,
            pltpu.SemaphoreType.DMA,
        ],
        compiler_params=pltpu.CompilerParams(needs_layout_passes=False),
    )
    def gather_kernel(idx_hbm, emb_hbm, tail_hbm, out_hbm,
                      idx_v, pos_v, bpos_v, cpos_v, chunk_v, ring_v, tail_v,
                      cnt_s, csem, rsem):
        wid = lax.axis_index("s") * NC + lax.axis_index("c")
        last = wid == NW - 1
        base = wid * SPAN
        himark = jnp.where(last, VOCAB, base + SPAN)
        nch = jnp.where(last, NCH_HI, NCH_LO)
        lane16 = jax.lax.iota(jnp.int32, 16)

        pltpu.sync_copy(idx_hbm, idx_v)
        cnt_s[0] = 0
        cnt_s[1] = 0

        def build(g, n):
            iv = idx_v[pl.ds(g * 16, 16)]
            mm = (iv >= base) & (iv < himark)
            inc = plsc.cumsum(jnp.where(mm, 1, 0))
            plsc.store_scatter(pos_v, [n + inc - 1], g * 16 + lane16, mask=mm)
            return n + inc[15]

        n_mine = lax.fori_loop(0, B // 16, build, 0)
        ngrp = (n_mine + 15) // 16

        def extract_hits(chunk_ref, clo, m_hits):
            def one_group(g2, _):
                valid = g2 * 16 + lane16 < m_hits
                cp = jnp.where(valid, cpos_v[pl.ds(g2 * 16, 16)], 0)
                iv = plsc.load_gather(idx_v, [cp])
                cols = jnp.where(valid, iv - clo, 0)
                valid_i = jnp.where(valid, 1, 0)
                for lane in range(16):
                    @pl.when(valid_i[lane] == 1)
                    def _():
                        slot = lax.rem(cnt_s[0], RING)
                        col16 = jnp.full((16,), cols[lane], jnp.int32)
                        for j in range(HIDDEN // 16):
                            vals = plsc.load_gather(
                                chunk_ref, [lane16 + j * 16, col16]
                            )
                            ring_v[slot, pl.ds(j * 16, 16)] = vals
                        pltpu.async_copy(
                            ring_v.at[pl.ds(slot, 1)],
                            out_hbm.at[pl.ds(cp[lane], 1)],
                            rsem,
                        )
                        cnt_s[0] = cnt_s[0] + 1
                        cnt_s[1] = cnt_s[1] + 1

                @pl.when(cnt_s[1] >= RING - 16)
                def _():
                    def drain(_, c):
                        pltpu.make_async_copy(
                            out_hbm.at[pl.ds(0, 1)],
                            ring_v.at[pl.ds(0, 1)],
                            rsem,
                        ).wait()
                        return c
                    lax.fori_loop(0, cnt_s[1], drain, 0)
                    cnt_s[1] = 0
                return 0

            lax.fori_loop(0, (m_hits + 15) // 16, one_group, 0)

        def scan_hits(clo, chi):
            def ga(g, m):
                valid = g * 16 + lane16 < n_mine
                pv = jnp.where(valid, pos_v[pl.ds(g * 16, 16)], 0)
                iv = plsc.load_gather(idx_v, [pv])
                mm = valid & (iv >= clo) & (iv < chi)
                inc = plsc.cumsum(jnp.where(mm, 1, 0))
                plsc.store_scatter(cpos_v, [m + inc - 1], pv, mask=mm)
                return m + inc[15]

            return lax.fori_loop(0, ngrp, ga, 0)

        BKT = 16 * CH
        off = 0
        for bkt in range(8):
            blo = base + bkt * BKT
            bhi = jnp.minimum(base + (bkt + 1) * BKT, himark)

            def gb(g, m, blo=blo, bhi=bhi):
                valid = g * 16 + lane16 < n_mine
                pv = jnp.where(valid, pos_v[pl.ds(g * 16, 16)], 0)
                iv = plsc.load_gather(idx_v, [pv])
                mm = valid & (iv >= blo) & (iv < bhi)
                inc = plsc.cumsum(jnp.where(mm, 1, 0))
                plsc.store_scatter(bpos_v, [m + inc - 1], pv, mask=mm)
                return m + inc[15]

            m_end = lax.fori_loop(0, ngrp, gb, off)
            cnt_s[2 + bkt] = off
            cnt_s[10 + bkt] = m_end - off
            off = ((m_end + 15) // 16) * 16

        def scan_hits_bucket(c, clo, chi):
            bkt = c // 16
            boff = cnt_s[2 + bkt]
            nb = cnt_s[10 + bkt]

            def ga(g, m):
                valid = g * 16 + lane16 < nb
                pv = jnp.where(valid, bpos_v[pl.ds(boff + g * 16, 16)], 0)
                iv = plsc.load_gather(idx_v, [pv])
                mm = valid & (iv >= clo) & (iv < chi)
                inc = plsc.cumsum(jnp.where(mm, 1, 0))
                plsc.store_scatter(cpos_v, [m + inc - 1], pv, mask=mm)
                return m + inc[15]

            return lax.fori_loop(0, (nb + 15) // 16, ga, 0)

        pltpu.async_copy(
            emb_hbm.at[:, pl.ds(pl.multiple_of(base, 128), CH)],
            chunk_v.at[0], csem,
        )

        def chunk_step(c, _):
            @pl.when(c + 1 < nch)
            def _():
                off = base + (c + 1) * CH
                pltpu.async_copy(
                    emb_hbm.at[:, pl.ds(pl.multiple_of(off, 128), CH)],
                    chunk_v.at[lax.rem(c + 1, 2)], csem,
                )
            pltpu.make_async_copy(
                emb_hbm.at[:, pl.ds(0, CH)], chunk_v.at[0], csem
            ).wait()
            clo = base + c * CH
            m_hits = scan_hits_bucket(c, clo, clo + CH)
            extract_hits(chunk_v.at[lax.rem(c, 2)], clo, m_hits)
            return 0

        lax.fori_loop(0, nch, chunk_step, 0)

        @pl.when(last)
        def _():
            pltpu.sync_copy(tail_hbm, tail_v)
            m_hits = scan_hits(TAIL0, VOCAB)
            extract_hits(tail_v, TAIL0, m_hits)

        def drain(_, c):
            pltpu.make_async_copy(
                out_hbm.at[pl.ds(0, 1)], ring_v.at[pl.ds(0, 1)], rsem
            ).wait()
            return c
        lax.fori_loop(0, cnt_s[1], drain, 0)

    return gather_kernel(idx, emb_t, tail_t)


def _tc_decode_t(x_t, w, b2):
    bm = 2048
    grid = (B // bm,)

    def body(w_ref, x_ref, b_ref, o_ref):
        xr = jnp.maximum(x_ref[...], 0.0)
        o_ref[...] = (
            jnp.dot(w_ref[...], xr, preferred_element_type=jnp.float32)
            + b_ref[...]
        )

    return pl.pallas_call(
        body,
        grid=grid,
        in_specs=[
            pl.BlockSpec((OUT, HIDDEN), lambda i: (0, 0)),
            pl.BlockSpec((HIDDEN, bm), lambda i: (0, i)),
            pl.BlockSpec((OUT, 1), lambda i: (0, 0)),
        ],
        out_specs=pl.BlockSpec((OUT, bm), lambda i: (0, i)),
        out_shape=jax.ShapeDtypeStruct((OUT, B), jnp.float32),
    )(w, x_t, b2)


def kernel(inp, hidden, emb, W, b):
    idx = inp.reshape(-1).astype(jnp.int32)
    emb_t = emb.T
    tail_t = lax.slice(emb_t, (0, TAIL0), (HIDDEN, VOCAB))
    x = _sc_gather_stream(idx, emb_t, tail_t)
    logits_t = _tc_decode_t(x.T, W, b.reshape(OUT, 1))
    return (logits_t.T, hidden)


# speedup vs baseline: 1.5790x; 1.0283x over previous
import functools

import jax
import jax.numpy as jnp
from jax import lax
from jax.experimental import pallas as pl
from jax.experimental.pallas import tpu as pltpu
from jax.experimental.pallas import tpu_sc as plsc

VOCAB = 1000000
HIDDEN = 64
OUT = 1000
B = 20480

NC = 2
NS = 16
NW = NC * NS

CH = 256
SPAN = 31232
NCH_LO = SPAN // CH
SPAN_HI = VOCAB - 64 - 31 * SPAN
NCH_HI = SPAN_HI // CH
TAIL0 = VOCAB - 64
RING = 32


def _sc_gather_stream(idx, emb_t, tail_t):
    mesh = plsc.VectorSubcoreMesh(core_axis_name="c", subcore_axis_name="s")

    @functools.partial(
        pl.kernel,
        mesh=mesh,
        out_type=jax.ShapeDtypeStruct((B, HIDDEN), jnp.float32),
        scratch_types=[
            pltpu.VMEM((B,), jnp.int32),
            pltpu.VMEM((B,), jnp.int32),
            pltpu.VMEM((B + 128,), jnp.int32),
            pltpu.VMEM((B,), jnp.int32),
            pltpu.VMEM((2, HIDDEN, CH), jnp.float32),
            pltpu.VMEM((RING, HIDDEN), jnp.float32),
            pltpu.VMEM((HIDDEN, 64), jnp.float32),
            pltpu.SMEM((24,), jnp.int32),
            pltpu.SemaphoreType.DMA,
            pltpu.SemaphoreType.DMA,
        ],
        compiler_params=pltpu.CompilerParams(needs_layout_passes=False),
    )
    def gather_kernel(idx_hbm, emb_hbm, tail_hbm, out_hbm,
                      idx_v, pos_v, bpos_v, cpos_v, chunk_v, ring_v, tail_v,
                      cnt_s, csem, rsem):
        wid = lax.axis_index("s") * NC + lax.axis_index("c")
        last = wid == NW - 1
        base = wid * SPAN
        himark = jnp.where(last, VOCAB, base + SPAN)
        nch = jnp.where(last, NCH_HI, NCH_LO)
        lane16 = jax.lax.iota(jnp.int32, 16)

        pltpu.async_copy(
            emb_hbm.at[:, pl.ds(pl.multiple_of(base, 128), CH)],
            chunk_v.at[0], csem,
        )
        pltpu.async_copy(
            emb_hbm.at[:, pl.ds(pl.multiple_of(base + CH, 128), CH)],
            chunk_v.at[1], csem,
        )

        pltpu.sync_copy(idx_hbm, idx_v)
        cnt_s[0] = 0
        cnt_s[1] = 0

        def build(g, n):
            iv = idx_v[pl.ds(g * 16, 16)]
            mm = (iv >= base) & (iv < himark)
            inc = plsc.cumsum(jnp.where(mm, 1, 0))
            plsc.store_scatter(pos_v, [n + inc - 1], g * 16 + lane16, mask=mm)
            return n + inc[15]

        n_mine = lax.fori_loop(0, B // 16, build, 0)
        ngrp = (n_mine + 15) // 16

        def extract_hits(chunk_ref, clo, m_hits):
            def one_group(g2, _):
                valid = g2 * 16 + lane16 < m_hits
                cp = jnp.where(valid, cpos_v[pl.ds(g2 * 16, 16)], 0)
                iv = plsc.load_gather(idx_v, [cp])
                cols = jnp.where(valid, iv - clo, 0)
                valid_i = jnp.where(valid, 1, 0)
                for lane in range(16):
                    @pl.when(valid_i[lane] == 1)
                    def _():
                        slot = lax.rem(cnt_s[0], RING)
                        col16 = jnp.full((16,), cols[lane], jnp.int32)
                        for j in range(HIDDEN // 16):
                            vals = plsc.load_gather(
                                chunk_ref, [lane16 + j * 16, col16]
                            )
                            ring_v[slot, pl.ds(j * 16, 16)] = vals
                        pltpu.async_copy(
                            ring_v.at[pl.ds(slot, 1)],
                            out_hbm.at[pl.ds(cp[lane], 1)],
                            rsem,
                        )
                        cnt_s[0] = cnt_s[0] + 1
                        cnt_s[1] = cnt_s[1] + 1

                @pl.when(cnt_s[1] >= RING - 16)
                def _():
                    def drain(_, c):
                        pltpu.make_async_copy(
                            out_hbm.at[pl.ds(0, 1)],
                            ring_v.at[pl.ds(0, 1)],
                            rsem,
                        ).wait()
                        return c
                    lax.fori_loop(0, cnt_s[1], drain, 0)
                    cnt_s[1] = 0
                return 0

            lax.fori_loop(0, (m_hits + 15) // 16, one_group, 0)

        def scan_hits(clo, chi):
            def ga(g, m):
                valid = g * 16 + lane16 < n_mine
                pv = jnp.where(valid, pos_v[pl.ds(g * 16, 16)], 0)
                iv = plsc.load_gather(idx_v, [pv])
                mm = valid & (iv >= clo) & (iv < chi)
                inc = plsc.cumsum(jnp.where(mm, 1, 0))
                plsc.store_scatter(cpos_v, [m + inc - 1], pv, mask=mm)
                return m + inc[15]

            return lax.fori_loop(0, ngrp, ga, 0)

        BKT = 16 * CH
        off = 0
        for bkt in range(8):
            blo = base + bkt * BKT
            bhi = jnp.minimum(base + (bkt + 1) * BKT, himark)

            def gb(g, m, blo=blo, bhi=bhi):
                valid = g * 16 + lane16 < n_mine
                pv = jnp.where(valid, pos_v[pl.ds(g * 16, 16)], 0)
                iv = plsc.load_gather(idx_v, [pv])
                mm = valid & (iv >= blo) & (iv < bhi)
                inc = plsc.cumsum(jnp.where(mm, 1, 0))
                plsc.store_scatter(bpos_v, [m + inc - 1], pv, mask=mm)
                return m + inc[15]

            m_end = lax.fori_loop(0, ngrp, gb, off)
            cnt_s[2 + bkt] = off
            cnt_s[10 + bkt] = m_end - off
            off = ((m_end + 15) // 16) * 16

        def scan_hits_bucket(c, clo, chi):
            bkt = c // 16
            boff = cnt_s[2 + bkt]
            nb = cnt_s[10 + bkt]

            def ga(g, m):
                valid = g * 16 + lane16 < nb
                pv = jnp.where(valid, bpos_v[pl.ds(boff + g * 16, 16)], 0)
                iv = plsc.load_gather(idx_v, [pv])
                mm = valid & (iv >= clo) & (iv < chi)
                inc = plsc.cumsum(jnp.where(mm, 1, 0))
                plsc.store_scatter(cpos_v, [m + inc - 1], pv, mask=mm)
                return m + inc[15]

            return lax.fori_loop(0, (nb + 15) // 16, ga, 0)

        def chunk_step(c, _):
            pltpu.make_async_copy(
                emb_hbm.at[:, pl.ds(0, CH)], chunk_v.at[0], csem
            ).wait()
            clo = base + c * CH
            m_hits = scan_hits_bucket(c, clo, clo + CH)
            extract_hits(chunk_v.at[lax.rem(c, 2)], clo, m_hits)
            @pl.when(c + 2 < nch)
            def _():
                off = base + (c + 2) * CH
                pltpu.async_copy(
                    emb_hbm.at[:, pl.ds(pl.multiple_of(off, 128), CH)],
                    chunk_v.at[lax.rem(c, 2)], csem,
                )
            return 0

        lax.fori_loop(0, nch, chunk_step, 0)

        @pl.when(last)
        def _():
            pltpu.sync_copy(tail_hbm, tail_v)
            m_hits = scan_hits(TAIL0, VOCAB)
            extract_hits(tail_v, TAIL0, m_hits)

        def drain(_, c):
            pltpu.make_async_copy(
                out_hbm.at[pl.ds(0, 1)], ring_v.at[pl.ds(0, 1)], rsem
            ).wait()
            return c
        lax.fori_loop(0, cnt_s[1], drain, 0)

    return gather_kernel(idx, emb_t, tail_t)


def _tc_decode_t(x, w, b2):
    bm = 2048
    grid = (B // bm,)

    def body(w_ref, x_ref, b_ref, o_ref):
        xr = jnp.maximum(x_ref[...].T, 0.0)
        o_ref[...] = (
            jnp.dot(w_ref[...], xr, preferred_element_type=jnp.float32)
            + b_ref[...]
        )

    return pl.pallas_call(
        body,
        grid=grid,
        in_specs=[
            pl.BlockSpec((OUT, HIDDEN), lambda i: (0, 0)),
            pl.BlockSpec((bm, HIDDEN), lambda i: (i, 0)),
            pl.BlockSpec((OUT, 1), lambda i: (0, 0)),
        ],
        out_specs=pl.BlockSpec((OUT, bm), lambda i: (0, i)),
        out_shape=jax.ShapeDtypeStruct((OUT, B), jnp.float32),
    )(w, x, b2)


def kernel(inp, hidden, emb, W, b):
    idx = inp.reshape(-1).astype(jnp.int32)
    emb_t = emb.T
    tail_t = lax.slice(emb_t, (0, TAIL0), (HIDDEN, VOCAB))
    x = _sc_gather_stream(idx, emb_t, tail_t)
    logits_t = _tc_decode_t(x, W, b.reshape(OUT, 1))
    return (logits_t.T, hidden)
